# baseline (speedup 1.0000x reference)
import functools

import jax
import jax.numpy as jnp
from jax import lax
from jax.experimental import pallas as pl
from jax.experimental.pallas import tpu as pltpu
from jax.experimental.pallas import tpu_sc as plsc

N = 10000
E = 320000
D_IN = 128
D_HID = 256
D_OUT = 128

NC = 2
NS = 16
NW = NC * NS
LANE = 128
HW = 64
N_PAD = 10240
ROWS_PER_TILE = N_PAD // NS
NCH = 80
E_PAD = NW * NCH * LANE
NBUF = 4


def _make_sc_agg(with_counts):
    mesh = plsc.VectorSubcoreMesh(core_axis_name="c", subcore_axis_name="s")
    out_type = jax.ShapeDtypeStruct((NC, 2, N_PAD, HW), jnp.float32)
    if with_counts:
        out_type = (out_type,
                    jax.ShapeDtypeStruct((NW, N_PAD), jnp.float32))
    scratch = [
        pltpu.VMEM((NCH, LANE), jnp.int32),
        pltpu.VMEM((NCH, LANE), jnp.int32),
        [pltpu.VMEM((LANE, HW), jnp.float32) for _ in range(NBUF)],
        pltpu.VMEM_SHARED((N_PAD, HW), jnp.float32),
        [pltpu.SemaphoreType.DMA for _ in range(NBUF)],
        [pltpu.SemaphoreType.DMA for _ in range(NBUF)],
    ]
    if with_counts:
        scratch.append(pltpu.VMEM((N_PAD,), jnp.float32))

    @functools.partial(
        pl.kernel, out_type=out_type, mesh=mesh, scratch_types=scratch,
        compiler_params=pltpu.CompilerParams(needs_layout_passes=False,
                                             use_tc_tiling_on_sc=False))
    def agg(t_lo, t_hi, src, dst, zeros, *rest):
        if with_counts:
            out, cnt_out, src_v, dst_v, bufs, acc, gsem, ssem, cnt_v = rest
        else:
            out, src_v, dst_v, bufs, acc, gsem, ssem = rest
            cnt_out = cnt_v = None
        c = lax.axis_index("c")
        s = lax.axis_index("s")
        wid = s * NC + c
        row0 = s * ROWS_PER_TILE
        pltpu.sync_copy(src.at[wid], src_v)
        pltpu.sync_copy(dst.at[wid], dst_v)
        if with_counts:
            @pl.loop(0, N_PAD // 16)
            def _(i):
                cnt_v[pl.ds(i * 16, 16)] = jnp.zeros((16,), jnp.float32)

        ones16 = jnp.ones((16,), jnp.float32)

        def count(j):
            for g in range(LANE // 16):
                d16 = dst_v[j, pl.ds(g * 16, 16)]
                plsc.addupdate_scatter(cnt_v, [d16], ones16)

        for h, table in enumerate((t_lo, t_hi)):
            do_cnt = with_counts and h == 0
            pltpu.sync_copy(zeros, acc.at[pl.ds(row0, ROWS_PER_TILE)])
            plsc.subcore_barrier()

            for b in range(NBUF):
                pltpu.async_copy(table.at[src_v.at[b]], bufs[b], gsem[b])

            @pl.loop(0, NCH, step=NBUF)
            def _(jj):
                for b in range(NBUF):
                    pltpu.make_async_copy(table.at[src_v.at[jj + b]],
                                          bufs[b], gsem[b]).wait()
                    pltpu.async_copy(bufs[b], acc.at[dst_v.at[jj + b]],
                                     ssem[b], add=True)
                    if do_cnt:
                        count(jj + b)
                for b in range(NBUF):
                    pltpu.make_async_copy(bufs[b], acc.at[dst_v.at[jj + b]],
                                          ssem[b]).wait()

                    @pl.when(jj + NBUF + b < NCH)
                    def _():
                        pltpu.async_copy(table.at[src_v.at[jj + NBUF + b]],
                                         bufs[b], gsem[b])

            plsc.subcore_barrier()
            pltpu.sync_copy(acc.at[pl.ds(row0, ROWS_PER_TILE)],
                            out.at[c, h, pl.ds(row0, ROWS_PER_TILE)])
        if with_counts:
            pltpu.sync_copy(cnt_v, cnt_out.at[wid])

    return agg


_sc_agg_l1 = _make_sc_agg(True)
_sc_agg_l2 = _make_sc_agg(False)

_BLK = 128
_GRID = N_PAD // _BLK


def _tc1_body(x_ref, s1a_ref, s1b_ref, cnt_ref,
              wl1_ref, wr1_ref, b1_ref, wl2_ref, h_ref, g2_ref, inv_ref):
    cnt = jnp.sum(cnt_ref[...], axis=0)
    inv = 1.0 / jnp.maximum(cnt, 1.0)
    inv_ref[...] = inv
    m = (s1a_ref[...] + s1b_ref[...]) * inv
    dn = (((1,), (1,)), ((), ()))
    pre = (lax.dot_general(m, wl1_ref[...], dn, preferred_element_type=jnp.float32)
           + lax.dot_general(x_ref[...], wr1_ref[...], dn,
                             preferred_element_type=jnp.float32)
           + b1_ref[...])
    h = jnp.maximum(pre, 0.0)
    h_ref[...] = h
    g2_ref[...] = lax.dot_general(h, wl2_ref[...], dn,
                                  preferred_element_type=jnp.float32)


def _tc2_body(h_ref, s2a_ref, s2b_ref, inv_ref, wr2_ref, b2_ref, o_ref):
    dn = (((1,), (1,)), ((), ()))
    o_ref[...] = ((s2a_ref[...] + s2b_ref[...]) * inv_ref[...]
                  + lax.dot_general(h_ref[...], wr2_ref[...], dn,
                                    preferred_element_type=jnp.float32)
                  + b2_ref[...])


def _row_spec(d):
    return pl.BlockSpec((_BLK, d), lambda i: (i, 0))


def _full_spec(r, c):
    return pl.BlockSpec((r, c), lambda i: (0, 0))


_tc1 = pl.pallas_call(
    _tc1_body,
    grid=(_GRID,),
    in_specs=[
        _row_spec(D_IN), _row_spec(D_IN), _row_spec(D_IN),
        pl.BlockSpec((NW, _BLK, 1), lambda i: (0, i, 0)),
        _full_spec(D_HID, D_IN), _full_spec(D_HID, D_IN), _full_spec(1, D_HID),
        _full_spec(D_OUT, D_HID),
    ],
    out_specs=[_row_spec(D_HID), _row_spec(D_OUT), _row_spec(1)],
    out_shape=[
        jax.ShapeDtypeStruct((N_PAD, D_HID), jnp.float32),
        jax.ShapeDtypeStruct((N_PAD, D_OUT), jnp.float32),
        jax.ShapeDtypeStruct((N_PAD, 1), jnp.float32),
    ],
)

_tc2 = pl.pallas_call(
    _tc2_body,
    grid=(_GRID,),
    in_specs=[
        _row_spec(D_HID), _row_spec(D_OUT), _row_spec(D_OUT), _row_spec(1),
        _full_spec(D_OUT, D_HID), _full_spec(1, D_OUT),
    ],
    out_specs=_row_spec(D_OUT),
    out_shape=jax.ShapeDtypeStruct((N_PAD, D_OUT), jnp.float32),
)


def _halves(a):
    return a[:, :HW], a[:, HW:]


def _cat(p):
    return jnp.concatenate([p[0], p[1]], axis=1)


def kernel(x, edge_index, Wl1, Wr1, b1, Wl2, Wr2, b2):
    src = edge_index[0]
    dst = edge_index[1]
    pad = E_PAD - E
    srcp = jnp.concatenate([src, jnp.zeros((pad,), jnp.int32)]).reshape(NW, NCH, LANE)
    dummy = N + jnp.arange(pad, dtype=jnp.int32) % (N_PAD - N)
    dstp = jnp.concatenate([dst, dummy]).reshape(NW, NCH, LANE)

    x_pad = jnp.zeros((N_PAD, D_IN), jnp.float32).at[:N].set(x)
    x_lo, x_hi = _halves(x_pad)
    zeros = jnp.zeros((ROWS_PER_TILE, HW), jnp.float32)

    p1, cnt = _sc_agg_l1(x_lo, x_hi, srcp, dstp, zeros)
    cnt3 = cnt.reshape(NW, N_PAD, 1)

    h, g2, inv = _tc1(x_pad, _cat(p1[0]), _cat(p1[1]), cnt3, Wl1, Wr1,
                      b1.reshape(1, D_HID), Wl2)

    g2_lo, g2_hi = _halves(g2)
    p2 = _sc_agg_l2(g2_lo, g2_hi, srcp, dstp, zeros)
    out = _tc2(h, _cat(p2[0]), _cat(p2[1]), inv, Wr2, b2.reshape(1, D_OUT))
    return out[:N]


# speedup vs baseline: 1.0124x; 1.0124x over previous
import functools

import jax
import jax.numpy as jnp
from jax import lax
from jax.experimental import pallas as pl
from jax.experimental.pallas import tpu as pltpu
from jax.experimental.pallas import tpu_sc as plsc

N = 10000
E = 320000
D_IN = 128
D_HID = 256
D_OUT = 128

NC = 2
NS = 16
NW = NC * NS
LANE = 320
HW = 64
N_PAD = 10240
ROWS_PER_TILE = N_PAD // NS
NCH = 32
E_PAD = NW * NCH * LANE
NBUF = 2


def _make_sc_agg(with_counts):
    mesh = plsc.VectorSubcoreMesh(core_axis_name="c", subcore_axis_name="s")
    out_type = jax.ShapeDtypeStruct((NC, 2, N_PAD, HW), jnp.float32)
    if with_counts:
        out_type = (out_type,
                    jax.ShapeDtypeStruct((NW, N_PAD), jnp.float32))
    scratch = [
        pltpu.VMEM((NCH, LANE), jnp.int32),
        pltpu.VMEM((NCH, LANE), jnp.int32),
        [pltpu.VMEM((LANE, HW), jnp.float32) for _ in range(NBUF)],
        pltpu.VMEM_SHARED((N_PAD, HW), jnp.float32),
        [pltpu.SemaphoreType.DMA for _ in range(NBUF)],
        [pltpu.SemaphoreType.DMA for _ in range(NBUF)],
    ]
    if with_counts:
        scratch.append(pltpu.VMEM((N_PAD,), jnp.float32))

    @functools.partial(
        pl.kernel, out_type=out_type, mesh=mesh, scratch_types=scratch,
        compiler_params=pltpu.CompilerParams(needs_layout_passes=False,
                                             use_tc_tiling_on_sc=False))
    def agg(t_lo, t_hi, src, dst, zeros, *rest):
        if with_counts:
            out, cnt_out, src_v, dst_v, bufs, acc, gsem, ssem, cnt_v = rest
        else:
            out, src_v, dst_v, bufs, acc, gsem, ssem = rest
            cnt_out = cnt_v = None
        c = lax.axis_index("c")
        s = lax.axis_index("s")
        wid = s * NC + c
        row0 = s * ROWS_PER_TILE
        pltpu.sync_copy(src.at[wid], src_v)
        pltpu.sync_copy(dst.at[wid], dst_v)
        if with_counts:
            @pl.loop(0, N_PAD // 16)
            def _(i):
                cnt_v[pl.ds(i * 16, 16)] = jnp.zeros((16,), jnp.float32)

        ones16 = jnp.ones((16,), jnp.float32)

        def count(j):
            for g in range(LANE // 16):
                d16 = dst_v[j, pl.ds(g * 16, 16)]
                plsc.addupdate_scatter(cnt_v, [d16], ones16)

        for h, table in enumerate((t_lo, t_hi)):
            do_cnt = with_counts and h == 0
            pltpu.sync_copy(zeros, acc.at[pl.ds(row0, ROWS_PER_TILE)])
            plsc.subcore_barrier()

            pltpu.async_copy(table.at[src_v.at[0]], bufs[0], gsem[0])

            @pl.loop(0, NCH, step=NBUF)
            def _(jj):
                for b in range(NBUF):
                    nxt = (b + 1) % NBUF

                    @pl.when(jj + b + 1 < NCH)
                    def _():
                        pltpu.async_copy(table.at[src_v.at[jj + b + 1]],
                                         bufs[nxt], gsem[nxt])

                    pltpu.make_async_copy(table.at[src_v.at[jj + b]],
                                          bufs[b], gsem[b]).wait()
                    pltpu.sync_copy(bufs[b], acc.at[dst_v.at[jj + b]], add=True)
                    if do_cnt:
                        count(jj + b)

            plsc.subcore_barrier()
            pltpu.sync_copy(acc.at[pl.ds(row0, ROWS_PER_TILE)],
                            out.at[c, h, pl.ds(row0, ROWS_PER_TILE)])
        if with_counts:
            pltpu.sync_copy(cnt_v, cnt_out.at[wid])

    return agg


_sc_agg_l1 = _make_sc_agg(True)
_sc_agg_l2 = _make_sc_agg(False)

_BLK = 128
_GRID = N_PAD // _BLK


def _tc1_body(x_ref, s1a_ref, s1b_ref, cnt_ref,
              wl1_ref, wr1_ref, b1_ref, wl2_ref, h_ref, g2_ref, inv_ref):
    cnt = jnp.sum(cnt_ref[...], axis=0)
    inv = 1.0 / jnp.maximum(cnt, 1.0)
    inv_ref[...] = inv
    m = (s1a_ref[...] + s1b_ref[...]) * inv
    dn = (((1,), (1,)), ((), ()))
    pre = (lax.dot_general(m, wl1_ref[...], dn, preferred_element_type=jnp.float32)
           + lax.dot_general(x_ref[...], wr1_ref[...], dn,
                             preferred_element_type=jnp.float32)
           + b1_ref[...])
    h = jnp.maximum(pre, 0.0)
    h_ref[...] = h
    g2_ref[...] = lax.dot_general(h, wl2_ref[...], dn,
                                  preferred_element_type=jnp.float32)


def _tc2_body(h_ref, s2a_ref, s2b_ref, inv_ref, wr2_ref, b2_ref, o_ref):
    dn = (((1,), (1,)), ((), ()))
    o_ref[...] = ((s2a_ref[...] + s2b_ref[...]) * inv_ref[...]
                  + lax.dot_general(h_ref[...], wr2_ref[...], dn,
                                    preferred_element_type=jnp.float32)
                  + b2_ref[...])


def _row_spec(d):
    return pl.BlockSpec((_BLK, d), lambda i: (i, 0))


def _full_spec(r, c):
    return pl.BlockSpec((r, c), lambda i: (0, 0))


_tc1 = pl.pallas_call(
    _tc1_body,
    grid=(_GRID,),
    in_specs=[
        _row_spec(D_IN), _row_spec(D_IN), _row_spec(D_IN),
        pl.BlockSpec((NW, _BLK, 1), lambda i: (0, i, 0)),
        _full_spec(D_HID, D_IN), _full_spec(D_HID, D_IN), _full_spec(1, D_HID),
        _full_spec(D_OUT, D_HID),
    ],
    out_specs=[_row_spec(D_HID), _row_spec(D_OUT), _row_spec(1)],
    out_shape=[
        jax.ShapeDtypeStruct((N_PAD, D_HID), jnp.float32),
        jax.ShapeDtypeStruct((N_PAD, D_OUT), jnp.float32),
        jax.ShapeDtypeStruct((N_PAD, 1), jnp.float32),
    ],
)

_tc2 = pl.pallas_call(
    _tc2_body,
    grid=(_GRID,),
    in_specs=[
        _row_spec(D_HID), _row_spec(D_OUT), _row_spec(D_OUT), _row_spec(1),
        _full_spec(D_OUT, D_HID), _full_spec(1, D_OUT),
    ],
    out_specs=_row_spec(D_OUT),
    out_shape=jax.ShapeDtypeStruct((N_PAD, D_OUT), jnp.float32),
)


def _halves(a):
    return a[:, :HW], a[:, HW:]


def _cat(p):
    return jnp.concatenate([p[0], p[1]], axis=1)


def kernel(x, edge_index, Wl1, Wr1, b1, Wl2, Wr2, b2):
    src = edge_index[0]
    dst = edge_index[1]
    pad = E_PAD - E
    srcp = jnp.concatenate([src, jnp.zeros((pad,), jnp.int32)]).reshape(NW, NCH, LANE)
    dummy = N + jnp.arange(pad, dtype=jnp.int32) % (N_PAD - N)
    dstp = jnp.concatenate([dst, dummy]).reshape(NW, NCH, LANE)

    x_pad = jnp.zeros((N_PAD, D_IN), jnp.float32).at[:N].set(x)
    x_lo, x_hi = _halves(x_pad)
    zeros = jnp.zeros((ROWS_PER_TILE, HW), jnp.float32)

    p1, cnt = _sc_agg_l1(x_lo, x_hi, srcp, dstp, zeros)
    cnt3 = cnt.reshape(NW, N_PAD, 1)

    h, g2, inv = _tc1(x_pad, _cat(p1[0]), _cat(p1[1]), cnt3, Wl1, Wr1,
                      b1.reshape(1, D_HID), Wl2)

    g2_lo, g2_hi = _halves(g2)
    p2 = _sc_agg_l2(g2_lo, g2_hi, srcp, dstp, zeros)
    out = _tc2(h, _cat(p2[0]), _cat(p2[1]), inv, Wr2, b2.reshape(1, D_OUT))
    return out[:N]


# speedup vs baseline: 1.1695x; 1.1551x over previous
import functools

import jax
import jax.numpy as jnp
from jax import lax
from jax.experimental import pallas as pl
from jax.experimental.pallas import tpu as pltpu
from jax.experimental.pallas import tpu_sc as plsc

N = 10000
E = 320000
D_IN = 128
D_HID = 256
D_OUT = 128

NC = 2
NS = 16
NW = NC * NS
LANE = 64
HW = 128
N_PAD = 10240
ROWS_PER_TILE = N_PAD // NS
NCH = 160
E_PAD = NW * NCH * LANE
NBUF = 2


def _make_sc_agg(with_counts, W=HW, lane=LANE, nch=NCH):
    npass = D_IN // W
    mesh = plsc.VectorSubcoreMesh(core_axis_name="c", subcore_axis_name="s")
    out_type = jax.ShapeDtypeStruct((NC, npass, N_PAD, W), jnp.float32)
    if with_counts:
        out_type = (out_type,
                    jax.ShapeDtypeStruct((NW, N_PAD), jnp.float32))
    scratch = [
        pltpu.VMEM((nch, lane), jnp.int32),
        pltpu.VMEM((nch, lane), jnp.int32),
        [pltpu.VMEM((lane, W), jnp.float32) for _ in range(NBUF)],
        pltpu.VMEM_SHARED((N_PAD, W), jnp.float32),
        [pltpu.SemaphoreType.DMA for _ in range(NBUF)],
    ]
    if with_counts:
        scratch.append(pltpu.VMEM((N_PAD,), jnp.float32))

    @functools.partial(
        pl.kernel, out_type=out_type, mesh=mesh, scratch_types=scratch,
        compiler_params=pltpu.CompilerParams(needs_layout_passes=False,
                                             use_tc_tiling_on_sc=False))
    def agg(*args):
        tables = args[:npass]
        src, dst, zeros = args[npass:npass + 3]
        rest = args[npass + 3:]
        if with_counts:
            out, cnt_out, src_v, dst_v, bufs, acc, gsem, cnt_v = rest
        else:
            out, src_v, dst_v, bufs, acc, gsem = rest
            cnt_out = cnt_v = None
        c = lax.axis_index("c")
        s = lax.axis_index("s")
        wid = s * NC + c
        row0 = s * ROWS_PER_TILE
        pltpu.sync_copy(src.at[wid], src_v)
        pltpu.sync_copy(dst.at[wid], dst_v)
        if with_counts:
            @pl.loop(0, N_PAD // 16)
            def _(i):
                cnt_v[pl.ds(i * 16, 16)] = jnp.zeros((16,), jnp.float32)

        ones16 = jnp.ones((16,), jnp.float32)

        def count(j):
            for g in range(lane // 16):
                d16 = dst_v[j, pl.ds(g * 16, 16)]
                plsc.addupdate_scatter(cnt_v, [d16], ones16)

        for h, table in enumerate(tables):
            do_cnt = with_counts and h == 0
            pltpu.sync_copy(zeros, acc.at[pl.ds(row0, ROWS_PER_TILE)])
            plsc.subcore_barrier()

            pltpu.async_copy(table.at[src_v.at[0]], bufs[0], gsem[0])

            @pl.loop(0, nch, step=NBUF)
            def _(jj):
                for b in range(NBUF):
                    nxt = (b + 1) % NBUF

                    @pl.when(jj + b + 1 < nch)
                    def _():
                        pltpu.async_copy(table.at[src_v.at[jj + b + 1]],
                                         bufs[nxt], gsem[nxt])

                    pltpu.make_async_copy(table.at[src_v.at[jj + b]],
                                          bufs[b], gsem[b]).wait()
                    pltpu.sync_copy(bufs[b], acc.at[dst_v.at[jj + b]], add=True)
                    if do_cnt:
                        count(jj + b)

            plsc.subcore_barrier()
            pltpu.sync_copy(acc.at[pl.ds(row0, ROWS_PER_TILE)],
                            out.at[c, h, pl.ds(row0, ROWS_PER_TILE)])
        if with_counts:
            pltpu.sync_copy(cnt_v, cnt_out.at[wid])

    return agg


_sc_agg_l1 = _make_sc_agg(True)
_sc_agg_l2 = _make_sc_agg(False)

_BLK = 128
_GRID = N_PAD // _BLK


def _tc1_body(x_ref, s1a_ref, s1b_ref, cnt_ref,
              wl1_ref, wr1_ref, b1_ref, wl2_ref, h_ref, g2_ref, inv_ref):
    cnt = jnp.sum(cnt_ref[...], axis=0)
    inv = 1.0 / jnp.maximum(cnt, 1.0)
    inv_ref[...] = inv
    m = (s1a_ref[...] + s1b_ref[...]) * inv
    dn = (((1,), (1,)), ((), ()))
    pre = (lax.dot_general(m, wl1_ref[...], dn, preferred_element_type=jnp.float32)
           + lax.dot_general(x_ref[...], wr1_ref[...], dn,
                             preferred_element_type=jnp.float32)
           + b1_ref[...])
    h = jnp.maximum(pre, 0.0)
    h_ref[...] = h
    g2_ref[...] = lax.dot_general(h, wl2_ref[...], dn,
                                  preferred_element_type=jnp.float32)


def _tc2_body(h_ref, s2a_ref, s2b_ref, inv_ref, wr2_ref, b2_ref, o_ref):
    dn = (((1,), (1,)), ((), ()))
    o_ref[...] = ((s2a_ref[...] + s2b_ref[...]) * inv_ref[...]
                  + lax.dot_general(h_ref[...], wr2_ref[...], dn,
                                    preferred_element_type=jnp.float32)
                  + b2_ref[...])


def _row_spec(d):
    return pl.BlockSpec((_BLK, d), lambda i: (i, 0))


def _full_spec(r, c):
    return pl.BlockSpec((r, c), lambda i: (0, 0))


_tc1 = pl.pallas_call(
    _tc1_body,
    grid=(_GRID,),
    in_specs=[
        _row_spec(D_IN), _row_spec(D_IN), _row_spec(D_IN),
        pl.BlockSpec((NW, _BLK, 1), lambda i: (0, i, 0)),
        _full_spec(D_HID, D_IN), _full_spec(D_HID, D_IN), _full_spec(1, D_HID),
        _full_spec(D_OUT, D_HID),
    ],
    out_specs=[_row_spec(D_HID), _row_spec(D_OUT), _row_spec(1)],
    out_shape=[
        jax.ShapeDtypeStruct((N_PAD, D_HID), jnp.float32),
        jax.ShapeDtypeStruct((N_PAD, D_OUT), jnp.float32),
        jax.ShapeDtypeStruct((N_PAD, 1), jnp.float32),
    ],
)

_tc2 = pl.pallas_call(
    _tc2_body,
    grid=(_GRID,),
    in_specs=[
        _row_spec(D_HID), _row_spec(D_OUT), _row_spec(D_OUT), _row_spec(1),
        _full_spec(D_OUT, D_HID), _full_spec(1, D_OUT),
    ],
    out_specs=_row_spec(D_OUT),
    out_shape=jax.ShapeDtypeStruct((N_PAD, D_OUT), jnp.float32),
)


def _split(a):
    return tuple(a[:, i * HW:(i + 1) * HW] for i in range(D_IN // HW))


def _cat(p):
    planes = [p[i] for i in range(p.shape[0])]
    return planes[0] if len(planes) == 1 else jnp.concatenate(planes, axis=1)


def kernel(x, edge_index, Wl1, Wr1, b1, Wl2, Wr2, b2):
    src = edge_index[0]
    dst = edge_index[1]
    pad = E_PAD - E
    srcp = jnp.concatenate([src, jnp.zeros((pad,), jnp.int32)]).reshape(NW, NCH, LANE)
    dummy = N + jnp.arange(pad, dtype=jnp.int32) % (N_PAD - N)
    dstp = jnp.concatenate([dst, dummy]).reshape(NW, NCH, LANE)

    x_pad = jnp.zeros((N_PAD, D_IN), jnp.float32).at[:N].set(x)
    zeros = jnp.zeros((ROWS_PER_TILE, HW), jnp.float32)

    p1, cnt = _sc_agg_l1(*_split(x_pad), srcp, dstp, zeros)
    cnt3 = cnt.reshape(NW, N_PAD, 1)

    h, g2, inv = _tc1(x_pad, _cat(p1[0]), _cat(p1[1]), cnt3, Wl1, Wr1,
                      b1.reshape(1, D_HID), Wl2)

    p2 = _sc_agg_l2(*_split(g2), srcp, dstp, zeros)
    out = _tc2(h, _cat(p2[0]), _cat(p2[1]), inv, Wr2, b2.reshape(1, D_OUT))
    return out[:N]


# speedup vs baseline: 1.8203x; 1.5565x over previous
import functools

import jax
import jax.numpy as jnp
from jax import lax
from jax.experimental import pallas as pl
from jax.experimental.pallas import tpu as pltpu
from jax.experimental.pallas import tpu_sc as plsc

N = 10000
E = 320000
D_IN = 128
D_HID = 256
D_OUT = 128

NC = 2
NS = 16
NW = NC * NS
LANE = 128
HW = 64
N_PAD = 10240
ROWS_PER_TILE = N_PAD // NS
NCH = 80
E_PAD = NW * NCH * LANE
NBUF = 2


def _make_sc_agg(with_counts, W=HW, lane=LANE, nch=NCH, stage_table=True):
    npass = D_IN // W
    mesh = plsc.VectorSubcoreMesh(core_axis_name="c", subcore_axis_name="s")
    out_type = jax.ShapeDtypeStruct((NC, npass, N_PAD, W), jnp.float32)
    if with_counts:
        out_type = (out_type,
                    jax.ShapeDtypeStruct((NW, N_PAD), jnp.float32))
    scratch = [
        pltpu.VMEM((nch, lane), jnp.int32),
        pltpu.VMEM((nch, lane), jnp.int32),
        [pltpu.VMEM((lane, W), jnp.float32) for _ in range(NBUF)],
        pltpu.VMEM_SHARED((N_PAD, W), jnp.float32),
        [pltpu.SemaphoreType.DMA for _ in range(NBUF)],
    ]
    if stage_table:
        scratch.append(pltpu.VMEM_SHARED((N_PAD, W), jnp.float32))
    if with_counts:
        scratch.append(pltpu.VMEM((N_PAD,), jnp.float32))

    @functools.partial(
        pl.kernel, out_type=out_type, mesh=mesh, scratch_types=scratch,
        compiler_params=pltpu.CompilerParams(needs_layout_passes=False,
                                             use_tc_tiling_on_sc=False))
    def agg(*args):
        tables = args[:npass]
        src, dst, zeros = args[npass:npass + 3]
        rest = args[npass + 3:]
        table_sh = None
        if with_counts and stage_table:
            out, cnt_out, src_v, dst_v, bufs, acc, gsem, table_sh, cnt_v = rest
        elif with_counts:
            out, cnt_out, src_v, dst_v, bufs, acc, gsem, cnt_v = rest
        elif stage_table:
            out, src_v, dst_v, bufs, acc, gsem, table_sh = rest
            cnt_out = cnt_v = None
        else:
            out, src_v, dst_v, bufs, acc, gsem = rest
            cnt_out = cnt_v = None
        c = lax.axis_index("c")
        s = lax.axis_index("s")
        wid = s * NC + c
        row0 = s * ROWS_PER_TILE
        pltpu.sync_copy(src.at[wid], src_v)
        pltpu.sync_copy(dst.at[wid], dst_v)
        if with_counts:
            @pl.loop(0, N_PAD // 16)
            def _(i):
                cnt_v[pl.ds(i * 16, 16)] = jnp.zeros((16,), jnp.float32)

        ones16 = jnp.ones((16,), jnp.float32)

        def count(j):
            for g in range(lane // 16):
                d16 = dst_v[j, pl.ds(g * 16, 16)]
                plsc.addupdate_scatter(cnt_v, [d16], ones16)

        for h, t_hbm in enumerate(tables):
            do_cnt = with_counts and h == 0
            pltpu.sync_copy(zeros, acc.at[pl.ds(row0, ROWS_PER_TILE)])
            if stage_table:
                pltpu.sync_copy(t_hbm.at[pl.ds(row0, ROWS_PER_TILE)],
                                table_sh.at[pl.ds(row0, ROWS_PER_TILE)])
                table = table_sh
            else:
                table = t_hbm
            plsc.subcore_barrier()

            pltpu.async_copy(table.at[src_v.at[0]], bufs[0], gsem[0])

            @pl.loop(0, nch, step=NBUF)
            def _(jj):
                for b in range(NBUF):
                    nxt = (b + 1) % NBUF

                    @pl.when(jj + b + 1 < nch)
                    def _():
                        pltpu.async_copy(table.at[src_v.at[jj + b + 1]],
                                         bufs[nxt], gsem[nxt])

                    pltpu.make_async_copy(table.at[src_v.at[jj + b]],
                                          bufs[b], gsem[b]).wait()
                    pltpu.sync_copy(bufs[b], acc.at[dst_v.at[jj + b]], add=True)
                    if do_cnt:
                        count(jj + b)

            plsc.subcore_barrier()
            pltpu.sync_copy(acc.at[pl.ds(row0, ROWS_PER_TILE)],
                            out.at[c, h, pl.ds(row0, ROWS_PER_TILE)])
        if with_counts:
            pltpu.sync_copy(cnt_v, cnt_out.at[wid])

    return agg


_sc_agg_l1 = _make_sc_agg(True)
_sc_agg_l2 = _make_sc_agg(False)

_BLK = 128
_GRID = N_PAD // _BLK


def _tc1_body(x_ref, s1a_ref, s1b_ref, cnt_ref,
              wl1_ref, wr1_ref, b1_ref, wl2_ref, h_ref, g2_ref, inv_ref):
    cnt = jnp.sum(cnt_ref[...], axis=0)
    inv = 1.0 / jnp.maximum(cnt, 1.0)
    inv_ref[...] = inv
    m = (s1a_ref[...] + s1b_ref[...]) * inv
    dn = (((1,), (1,)), ((), ()))
    pre = (lax.dot_general(m, wl1_ref[...], dn, preferred_element_type=jnp.float32)
           + lax.dot_general(x_ref[...], wr1_ref[...], dn,
                             preferred_element_type=jnp.float32)
           + b1_ref[...])
    h = jnp.maximum(pre, 0.0)
    h_ref[...] = h
    g2_ref[...] = lax.dot_general(h, wl2_ref[...], dn,
                                  preferred_element_type=jnp.float32)


def _tc2_body(h_ref, s2a_ref, s2b_ref, inv_ref, wr2_ref, b2_ref, o_ref):
    dn = (((1,), (1,)), ((), ()))
    o_ref[...] = ((s2a_ref[...] + s2b_ref[...]) * inv_ref[...]
                  + lax.dot_general(h_ref[...], wr2_ref[...], dn,
                                    preferred_element_type=jnp.float32)
                  + b2_ref[...])


def _row_spec(d):
    return pl.BlockSpec((_BLK, d), lambda i: (i, 0))


def _full_spec(r, c):
    return pl.BlockSpec((r, c), lambda i: (0, 0))


_tc1 = pl.pallas_call(
    _tc1_body,
    grid=(_GRID,),
    in_specs=[
        _row_spec(D_IN), _row_spec(D_IN), _row_spec(D_IN),
        pl.BlockSpec((NW, _BLK, 1), lambda i: (0, i, 0)),
        _full_spec(D_HID, D_IN), _full_spec(D_HID, D_IN), _full_spec(1, D_HID),
        _full_spec(D_OUT, D_HID),
    ],
    out_specs=[_row_spec(D_HID), _row_spec(D_OUT), _row_spec(1)],
    out_shape=[
        jax.ShapeDtypeStruct((N_PAD, D_HID), jnp.float32),
        jax.ShapeDtypeStruct((N_PAD, D_OUT), jnp.float32),
        jax.ShapeDtypeStruct((N_PAD, 1), jnp.float32),
    ],
)

_tc2 = pl.pallas_call(
    _tc2_body,
    grid=(_GRID,),
    in_specs=[
        _row_spec(D_HID), _row_spec(D_OUT), _row_spec(D_OUT), _row_spec(1),
        _full_spec(D_OUT, D_HID), _full_spec(1, D_OUT),
    ],
    out_specs=_row_spec(D_OUT),
    out_shape=jax.ShapeDtypeStruct((N_PAD, D_OUT), jnp.float32),
)


def _split(a):
    return tuple(a[:, i * HW:(i + 1) * HW] for i in range(D_IN // HW))


def _cat(p):
    planes = [p[i] for i in range(p.shape[0])]
    return planes[0] if len(planes) == 1 else jnp.concatenate(planes, axis=1)


def kernel(x, edge_index, Wl1, Wr1, b1, Wl2, Wr2, b2):
    src = edge_index[0]
    dst = edge_index[1]
    pad = E_PAD - E
    srcp = jnp.concatenate([src, jnp.zeros((pad,), jnp.int32)]).reshape(NW, NCH, LANE)
    dummy = N + jnp.arange(pad, dtype=jnp.int32) % (N_PAD - N)
    dstp = jnp.concatenate([dst, dummy]).reshape(NW, NCH, LANE)

    x_pad = jnp.zeros((N_PAD, D_IN), jnp.float32).at[:N].set(x)
    zeros = jnp.zeros((ROWS_PER_TILE, HW), jnp.float32)

    p1, cnt = _sc_agg_l1(*_split(x_pad), srcp, dstp, zeros)
    cnt3 = cnt.reshape(NW, N_PAD, 1)

    h, g2, inv = _tc1(x_pad, _cat(p1[0]), _cat(p1[1]), cnt3, Wl1, Wr1,
                      b1.reshape(1, D_HID), Wl2)

    p2 = _sc_agg_l2(*_split(g2), srcp, dstp, zeros)
    out = _tc2(h, _cat(p2[0]), _cat(p2[1]), inv, Wr2, b2.reshape(1, D_OUT))
    return out[:N]
